# Initial kernel scaffold; baseline (speedup 1.0000x reference)
import jax
import jax.numpy as jnp
from jax.experimental import pallas as pl


def kernel(h, e, W_enc, b_enc, Ws, bs, gammas, betas, W_out, b_out, edge_index):
    raise NotImplementedError("write your pallas kernel here")


# speedup vs baseline: 2.5501x; 2.5501x over previous
import functools

import jax
import jax.numpy as jnp
from jax import lax
from jax.experimental import pallas as pl
from jax.experimental.pallas import tpu as pltpu
from jax.experimental.pallas import tpu_sc as plsc

N = 10000
E = 320000
H = 128
W16 = 16

NC = 2
NS = 16
NT = NC * NS
K = 128
CH = 80
EPAD = NT * CH * K
NR = 10240
RPT = NR // NS

_mesh = plsc.VectorSubcoreMesh(
    core_axis_name="c", subcore_axis_name="s", num_cores=NC, num_subcores=NS
)


@functools.partial(
    pl.kernel,
    out_type=[jax.ShapeDtypeStruct((NC, NR, H), jnp.float32)],
    mesh=_mesh,
    scratch_types=[
        pltpu.VMEM((K,), jnp.int32),
        pltpu.VMEM((K,), jnp.int32),
        pltpu.VMEM((K, H), jnp.float32),
        pltpu.VMEM_SHARED((NR, H), jnp.float32),
        pltpu.SemaphoreType.DMA,
    ],
)
def _agg(x_hbm, srcp, dstp, zrow, out_hbm, src_v, dst_v, rows, acc, sem):
    c = lax.axis_index("c")
    s = lax.axis_index("s")
    wid = s * NC + c
    pltpu.sync_copy(zrow.at[pl.ds(0, K)], rows)
    for q in range(RPT // K):
        pltpu.sync_copy(rows, acc.at[pl.ds(s * RPT + q * K, K)])
    plsc.subcore_barrier()

    def step(j, carry):
        pltpu.sync_copy(srcp.at[wid * CH + j], src_v)
        pltpu.sync_copy(dstp.at[wid * CH + j], dst_v)
        pltpu.async_copy(x_hbm.at[src_v], rows, sem).wait()
        pltpu.sync_copy(rows, acc.at[dst_v], add=True)
        return carry

    lax.fori_loop(0, CH, step, 0)
    plsc.subcore_barrier()
    for q in range(RPT // K):
        sl = pl.ds(s * RPT + q * K, K)
        pltpu.sync_copy(acc.at[sl], rows)
        pltpu.sync_copy(rows, out_hbm.at[c, sl])


@functools.partial(
    pl.kernel,
    out_type=[jax.ShapeDtypeStruct((NC, NR, H), jnp.float32)],
    mesh=_mesh,
    scratch_types=[
        pltpu.VMEM((K,), jnp.int32),
        pltpu.VMEM((K, H), jnp.float32),
        pltpu.VMEM_SHARED((NR, H), jnp.float32),
    ],
)
def _deg(dstp, z16, o16, out_hbm, dst_v, rows, acc):
    c = lax.axis_index("c")
    s = lax.axis_index("s")
    wid = s * NC + c
    pltpu.sync_copy(z16.at[pl.ds(0, K)], rows)
    for q in range(RPT // K):
        pltpu.sync_copy(rows, acc.at[pl.ds(s * RPT + q * K, K)])
    plsc.subcore_barrier()
    pltpu.sync_copy(o16.at[pl.ds(0, K)], rows)

    def step(j, carry):
        pltpu.sync_copy(dstp.at[wid * CH + j], dst_v)
        pltpu.sync_copy(rows, acc.at[dst_v], add=True)
        return carry

    lax.fori_loop(0, CH, step, 0)
    plsc.subcore_barrier()
    for q in range(RPT // K):
        sl = pl.ds(s * RPT + q * K, K)
        pltpu.sync_copy(acc.at[sl], rows)
        pltpu.sync_copy(rows, out_hbm.at[c, sl])


def _enc_body(h_ref, w_ref, b_ref, o_ref):
    o_ref[...] = (
        jnp.dot(h_ref[...], w_ref[...], preferred_element_type=jnp.float32)
        + b_ref[...]
    )


def _layer_body(p_ref, degp_ref, x_ref, w_ref, b_ref, g_ref, be_ref, o_ref):
    agg = p_ref[0, :N, :] + p_ref[1, :N, :]
    deg = jnp.maximum(degp_ref[0, :N, :1] + degp_ref[1, :N, :1], 1.0)
    y = (
        jnp.dot(agg / deg, w_ref[...], preferred_element_type=jnp.float32)
        + b_ref[...]
    )
    mu = jnp.mean(y, axis=0, keepdims=True)
    var = jnp.mean((y - mu) ** 2, axis=0, keepdims=True)
    yn = (y - mu) * lax.rsqrt(var + 1e-5) * g_ref[...] + be_ref[...]
    o_ref[...] = x_ref[...] + jnp.maximum(yn, 0.0)


def _out_body(x_ref, w_ref, b_ref, o_ref):
    o_ref[...] = (
        jnp.dot(x_ref[...], w_ref[...], preferred_element_type=jnp.float32)
        + b_ref[...]
    )


def kernel(h, e, W_enc, b_enc, Ws, bs, gammas, betas, W_out, b_out, edge_index):
    del e
    src = edge_index[0]
    dst = edge_index[1]
    srcp = jnp.concatenate(
        [src, jnp.zeros((EPAD - E,), jnp.int32)]).reshape(NT * CH, K)
    dstp = jnp.concatenate(
        [dst, jnp.full((EPAD - E,), N, jnp.int32)]).reshape(NT * CH, K)
    zrow = jnp.zeros((K, H), jnp.float32)

    x = pl.pallas_call(
        _enc_body, out_shape=jax.ShapeDtypeStruct((N, H), jnp.float32)
    )(h, W_enc, b_enc.reshape(1, H))

    (degp,) = _deg(dstp, zrow, jnp.ones((K, H), jnp.float32))
    srcp, degp = lax.optimization_barrier((srcp, degp))
    for i in range(Ws.shape[0]):
        (p,) = _agg(x, srcp, dstp, zrow)
        x = pl.pallas_call(
            _layer_body, out_shape=jax.ShapeDtypeStruct((N, H), jnp.float32)
        )(
            p, degp, x, Ws[i], bs[i].reshape(1, H),
            gammas[i].reshape(1, H), betas[i].reshape(1, H),
        )

    C = W_out.shape[1]
    return pl.pallas_call(
        _out_body, out_shape=jax.ShapeDtypeStruct((N, C), jnp.float32)
    )(x, W_out, b_out.reshape(1, C))


# speedup vs baseline: 3.1299x; 1.2273x over previous
import functools

import jax
import jax.numpy as jnp
from jax import lax
from jax.experimental import pallas as pl
from jax.experimental.pallas import tpu as pltpu
from jax.experimental.pallas import tpu_sc as plsc

N = 10000
E = 320000
H = 128
W16 = 16

NC = 2
NS = 16
NT = NC * NS
K = 128
CH = 80
EPAD = NT * CH * K
NR = 10240
RPT = NR // NS

_mesh = plsc.VectorSubcoreMesh(
    core_axis_name="c", subcore_axis_name="s", num_cores=NC, num_subcores=NS
)


@functools.partial(
    pl.kernel,
    out_type=[jax.ShapeDtypeStruct((NC, NR, H), jnp.float32)],
    mesh=_mesh,
    scratch_types=[
        pltpu.VMEM((2, K), jnp.int32),
        pltpu.VMEM((2, K), jnp.int32),
        pltpu.VMEM((2, K, H), jnp.float32),
        pltpu.VMEM_SHARED((NR, H), jnp.float32),
        pltpu.SemaphoreType.DMA,
        pltpu.SemaphoreType.DMA,
    ],
)
def _agg(x_hbm, srcp, dstp, zrow, out_hbm, src_v, dst_v, rows, acc, sem0, sem1):
    c = lax.axis_index("c")
    s = lax.axis_index("s")
    base = (s * NC + c) * CH
    pltpu.sync_copy(zrow.at[pl.ds(0, K)], rows.at[0])
    for q in range(RPT // K):
        pltpu.sync_copy(rows.at[0], acc.at[pl.ds(s * RPT + q * K, K)])
    plsc.subcore_barrier()

    sems = (sem0, sem1)
    pltpu.sync_copy(srcp.at[base], src_v.at[0])
    pltpu.sync_copy(dstp.at[base], dst_v.at[0])
    pltpu.async_copy(x_hbm.at[src_v.at[0]], rows.at[0], sems[0])

    @pl.loop(0, CH, step=2)
    def _chunks(g):
        for b in range(2):
            j = g + b
            nb = 1 - b

            @pl.when(j + 1 < CH)
            def _prefetch():
                pltpu.sync_copy(srcp.at[base + j + 1], src_v.at[nb])
                pltpu.sync_copy(dstp.at[base + j + 1], dst_v.at[nb])
                pltpu.async_copy(x_hbm.at[src_v.at[nb]], rows.at[nb], sems[nb])

            pltpu.make_async_copy(zrow, rows.at[b], sems[b]).wait()
            pltpu.sync_copy(rows.at[b], acc.at[dst_v.at[b]], add=True)

    plsc.subcore_barrier()
    for q in range(RPT // K):
        sl = pl.ds(s * RPT + q * K, K)
        pltpu.sync_copy(acc.at[sl], rows.at[0])
        pltpu.sync_copy(rows.at[0], out_hbm.at[c, sl])


@functools.partial(
    pl.kernel,
    out_type=[jax.ShapeDtypeStruct((NC, NR, H), jnp.float32)],
    mesh=_mesh,
    scratch_types=[
        pltpu.VMEM((K,), jnp.int32),
        pltpu.VMEM((K, H), jnp.float32),
        pltpu.VMEM_SHARED((NR, H), jnp.float32),
    ],
)
def _deg(dstp, z16, o16, out_hbm, dst_v, rows, acc):
    c = lax.axis_index("c")
    s = lax.axis_index("s")
    wid = s * NC + c
    pltpu.sync_copy(z16.at[pl.ds(0, K)], rows)
    for q in range(RPT // K):
        pltpu.sync_copy(rows, acc.at[pl.ds(s * RPT + q * K, K)])
    plsc.subcore_barrier()
    pltpu.sync_copy(o16.at[pl.ds(0, K)], rows)

    def step(j, carry):
        pltpu.sync_copy(dstp.at[wid * CH + j], dst_v)
        pltpu.sync_copy(rows, acc.at[dst_v], add=True)
        return carry

    lax.fori_loop(0, CH, step, 0)
    plsc.subcore_barrier()
    for q in range(RPT // K):
        sl = pl.ds(s * RPT + q * K, K)
        pltpu.sync_copy(acc.at[sl], rows)
        pltpu.sync_copy(rows, out_hbm.at[c, sl])


def _enc_body(h_ref, w_ref, b_ref, o_ref):
    o_ref[...] = (
        jnp.dot(h_ref[...], w_ref[...], preferred_element_type=jnp.float32)
        + b_ref[...]
    )


def _layer_body(p_ref, degp_ref, x_ref, w_ref, b_ref, g_ref, be_ref, o_ref):
    agg = p_ref[0, :N, :] + p_ref[1, :N, :]
    deg = jnp.maximum(degp_ref[0, :N, :1] + degp_ref[1, :N, :1], 1.0)
    y = (
        jnp.dot(agg / deg, w_ref[...], preferred_element_type=jnp.float32)
        + b_ref[...]
    )
    mu = jnp.mean(y, axis=0, keepdims=True)
    var = jnp.mean((y - mu) ** 2, axis=0, keepdims=True)
    yn = (y - mu) * lax.rsqrt(var + 1e-5) * g_ref[...] + be_ref[...]
    o_ref[...] = x_ref[...] + jnp.maximum(yn, 0.0)


def _out_body(x_ref, w_ref, b_ref, o_ref):
    o_ref[...] = (
        jnp.dot(x_ref[...], w_ref[...], preferred_element_type=jnp.float32)
        + b_ref[...]
    )


def kernel(h, e, W_enc, b_enc, Ws, bs, gammas, betas, W_out, b_out, edge_index):
    del e
    src = edge_index[0]
    dst = edge_index[1]
    srcp = jnp.concatenate(
        [src, jnp.zeros((EPAD - E,), jnp.int32)]).reshape(NT * CH, K)
    dstp = jnp.concatenate(
        [dst, jnp.full((EPAD - E,), N, jnp.int32)]).reshape(NT * CH, K)
    zrow = jnp.zeros((K, H), jnp.float32)

    x = pl.pallas_call(
        _enc_body, out_shape=jax.ShapeDtypeStruct((N, H), jnp.float32)
    )(h, W_enc, b_enc.reshape(1, H))

    (degp,) = _deg(dstp, zrow, jnp.ones((K, H), jnp.float32))
    srcp, degp = lax.optimization_barrier((srcp, degp))
    for i in range(Ws.shape[0]):
        (p,) = _agg(x, srcp, dstp, zrow)
        x = pl.pallas_call(
            _layer_body, out_shape=jax.ShapeDtypeStruct((N, H), jnp.float32)
        )(
            p, degp, x, Ws[i], bs[i].reshape(1, H),
            gammas[i].reshape(1, H), betas[i].reshape(1, H),
        )

    C = W_out.shape[1]
    return pl.pallas_call(
        _out_body, out_shape=jax.ShapeDtypeStruct((N, C), jnp.float32)
    )(x, W_out, b_out.reshape(1, C))


# speedup vs baseline: 3.1677x; 1.0121x over previous
import functools

import jax
import jax.numpy as jnp
from jax import lax
from jax.experimental import pallas as pl
from jax.experimental.pallas import tpu as pltpu
from jax.experimental.pallas import tpu_sc as plsc

N = 10000
E = 320000
H = 128
W16 = 16

NC = 2
NS = 16
NT = NC * NS
K = 128
CH = 80
EPAD = NT * CH * K
NR = 10240
RPT = NR // NS

_mesh = plsc.VectorSubcoreMesh(
    core_axis_name="c", subcore_axis_name="s", num_cores=NC, num_subcores=NS
)


@functools.partial(
    pl.kernel,
    out_type=[jax.ShapeDtypeStruct((NC, NR, H), jnp.float32)],
    mesh=_mesh,
    scratch_types=[
        pltpu.VMEM((CH // 2, K), jnp.int32),
        pltpu.VMEM((CH // 2, K), jnp.int32),
        pltpu.VMEM((2, K, H), jnp.float32),
        pltpu.VMEM_SHARED((NR, H), jnp.float32),
        pltpu.SemaphoreType.DMA,
        pltpu.SemaphoreType.DMA,
    ],
)
def _agg(x_hbm, srcp, dstp, zrow, out_hbm, src_all, dst_all, rows, acc, sem0, sem1):
    c = lax.axis_index("c")
    s = lax.axis_index("s")
    base = (s * NC + c) * CH
    pltpu.sync_copy(zrow.at[pl.ds(0, K)], rows.at[0])
    for q in range(RPT // K):
        pltpu.sync_copy(rows.at[0], acc.at[pl.ds(s * RPT + q * K, K)])
    plsc.subcore_barrier()

    sems = (sem0, sem1)
    CH2 = CH // 2
    for h in range(2):
        pltpu.sync_copy(srcp.at[pl.ds(base + h * CH2, CH2)], src_all)
        pltpu.sync_copy(dstp.at[pl.ds(base + h * CH2, CH2)], dst_all)
        pltpu.async_copy(x_hbm.at[src_all.at[0]], rows.at[0], sems[0])

        @pl.loop(0, CH2, step=2)
        def _chunks(g):
            for b in range(2):
                j = g + b
                nb = 1 - b

                @pl.when(j + 1 < CH2)
                def _prefetch():
                    pltpu.async_copy(
                        x_hbm.at[src_all.at[j + 1]], rows.at[nb], sems[nb]
                    )

                pltpu.make_async_copy(zrow, rows.at[b], sems[b]).wait()
                pltpu.sync_copy(rows.at[b], acc.at[dst_all.at[j]], add=True)

    plsc.subcore_barrier()
    for q in range(RPT // K):
        sl = pl.ds(s * RPT + q * K, K)
        pltpu.sync_copy(acc.at[sl], rows.at[0])
        pltpu.sync_copy(rows.at[0], out_hbm.at[c, sl])


@functools.partial(
    pl.kernel,
    out_type=[jax.ShapeDtypeStruct((NC, NR, H), jnp.float32)],
    mesh=_mesh,
    scratch_types=[
        pltpu.VMEM((K,), jnp.int32),
        pltpu.VMEM((K, H), jnp.float32),
        pltpu.VMEM_SHARED((NR, H), jnp.float32),
    ],
)
def _deg(dstp, z16, o16, out_hbm, dst_v, rows, acc):
    c = lax.axis_index("c")
    s = lax.axis_index("s")
    wid = s * NC + c
    pltpu.sync_copy(z16.at[pl.ds(0, K)], rows)
    for q in range(RPT // K):
        pltpu.sync_copy(rows, acc.at[pl.ds(s * RPT + q * K, K)])
    plsc.subcore_barrier()
    pltpu.sync_copy(o16.at[pl.ds(0, K)], rows)

    def step(j, carry):
        pltpu.sync_copy(dstp.at[wid * CH + j], dst_v)
        pltpu.sync_copy(rows, acc.at[dst_v], add=True)
        return carry

    lax.fori_loop(0, CH, step, 0)
    plsc.subcore_barrier()
    for q in range(RPT // K):
        sl = pl.ds(s * RPT + q * K, K)
        pltpu.sync_copy(acc.at[sl], rows)
        pltpu.sync_copy(rows, out_hbm.at[c, sl])


def _enc_body(h_ref, w_ref, b_ref, o_ref):
    o_ref[...] = (
        jnp.dot(h_ref[...], w_ref[...], preferred_element_type=jnp.float32)
        + b_ref[...]
    )


def _layer_body(p_ref, degp_ref, x_ref, w_ref, b_ref, g_ref, be_ref, o_ref):
    agg = p_ref[0, :N, :] + p_ref[1, :N, :]
    deg = jnp.maximum(degp_ref[0, :N, :1] + degp_ref[1, :N, :1], 1.0)
    y = (
        jnp.dot(agg / deg, w_ref[...], preferred_element_type=jnp.float32)
        + b_ref[...]
    )
    mu = jnp.mean(y, axis=0, keepdims=True)
    var = jnp.mean((y - mu) ** 2, axis=0, keepdims=True)
    yn = (y - mu) * lax.rsqrt(var + 1e-5) * g_ref[...] + be_ref[...]
    o_ref[...] = x_ref[...] + jnp.maximum(yn, 0.0)


def _out_body(x_ref, w_ref, b_ref, o_ref):
    o_ref[...] = (
        jnp.dot(x_ref[...], w_ref[...], preferred_element_type=jnp.float32)
        + b_ref[...]
    )


def kernel(h, e, W_enc, b_enc, Ws, bs, gammas, betas, W_out, b_out, edge_index):
    del e
    src = edge_index[0]
    dst = edge_index[1]
    srcp = jnp.concatenate(
        [src, jnp.zeros((EPAD - E,), jnp.int32)]).reshape(NT * CH, K)
    dstp = jnp.concatenate(
        [dst, jnp.full((EPAD - E,), N, jnp.int32)]).reshape(NT * CH, K)
    zrow = jnp.zeros((K, H), jnp.float32)

    x = pl.pallas_call(
        _enc_body, out_shape=jax.ShapeDtypeStruct((N, H), jnp.float32)
    )(h, W_enc, b_enc.reshape(1, H))

    (degp,) = _deg(dstp, zrow, jnp.ones((K, H), jnp.float32))
    srcp, degp = lax.optimization_barrier((srcp, degp))
    for i in range(Ws.shape[0]):
        (p,) = _agg(x, srcp, dstp, zrow)
        x = pl.pallas_call(
            _layer_body, out_shape=jax.ShapeDtypeStruct((N, H), jnp.float32)
        )(
            p, degp, x, Ws[i], bs[i].reshape(1, H),
            gammas[i].reshape(1, H), betas[i].reshape(1, H),
        )

    C = W_out.shape[1]
    return pl.pallas_call(
        _out_body, out_shape=jax.ShapeDtypeStruct((N, C), jnp.float32)
    )(x, W_out, b_out.reshape(1, C))
